# Initial kernel scaffold; baseline (speedup 1.0000x reference)
import jax
import jax.numpy as jnp
from jax.experimental import pallas as pl


def kernel(predicted_locs, predicted_scores, boxes, labels, priors_cxcy):
    raise NotImplementedError("write your pallas kernel here")


# speedup vs baseline: 45.7415x; 45.7415x over previous
import jax
import jax.numpy as jnp
from jax.experimental import pallas as pl

_THRESHOLD = 0.5
_NEG_POS_RATIO = 3.0
_ALPHA = 1.0


def _stage1_body(locs_ref, scores_ref, boxes_ref, labels_ref, priors_ref,
                 conf_ref, stats_ref):
    n_obj = boxes_ref.shape[1]
    p = priors_ref.shape[1]

    prior = priors_ref[...]
    pcx = prior[0:1]
    pcy = prior[1:2]
    pw = prior[2:3]
    ph = prior[3:4]
    px0 = pcx - pw / 2.0
    py0 = pcy - ph / 2.0
    px1 = pcx + pw / 2.0
    py1 = pcy + ph / 2.0

    boxes_b = boxes_ref[0]
    bx0 = boxes_b[:, 0:1]
    by0 = boxes_b[:, 1:2]
    bx1 = boxes_b[:, 2:3]
    by1 = boxes_b[:, 3:4]

    iw = jnp.maximum(jnp.minimum(bx1, px1) - jnp.maximum(bx0, px0), 0.0)
    ih = jnp.maximum(jnp.minimum(by1, py1) - jnp.maximum(by0, py0), 0.0)
    inter = iw * ih
    a1 = (bx1 - bx0) * (by1 - by0)
    a2 = (px1 - px0) * (py1 - py0)
    ov = inter / (a1 + a2 - inter)

    jrow = jax.lax.broadcasted_iota(jnp.int32, (n_obj, p), 0)
    crow = jax.lax.broadcasted_iota(jnp.int32, (n_obj, p), 1)

    ofp = jnp.max(ov, axis=0, keepdims=True)
    obj_fp = jnp.min(jnp.where(ov >= ofp, jrow, n_obj),
                     axis=0, keepdims=True)
    rmax = jnp.max(ov, axis=1, keepdims=True)
    pfo = jnp.min(jnp.where(ov >= rmax, crow, p),
                  axis=1, keepdims=True)
    matched = crow == pfo
    j_asgn = jnp.max(jnp.where(matched, jrow, -1),
                     axis=0, keepdims=True)
    forced = j_asgn >= 0
    obj_fp = jnp.where(forced, j_asgn, obj_fp)
    ofp = jnp.where(forced, 1.0, ofp)

    onehot = obj_fp == jrow
    labels_b = labels_ref[0]
    lab = jnp.max(jnp.where(onehot, labels_b, 0), axis=0, keepdims=True)
    bmx0 = jnp.sum(jnp.where(onehot, bx0, 0.0), axis=0, keepdims=True)
    bmy0 = jnp.sum(jnp.where(onehot, by0, 0.0), axis=0, keepdims=True)
    bmx1 = jnp.sum(jnp.where(onehot, bx1, 0.0), axis=0, keepdims=True)
    bmy1 = jnp.sum(jnp.where(onehot, by1, 0.0), axis=0, keepdims=True)

    tc = jnp.where(ofp < _THRESHOLD, 0, lab)
    posm = tc > 0
    posf = jnp.where(posm, 1.0, 0.0)
    n_pos = jnp.sum(posf)

    cx = (bmx0 + bmx1) / 2.0
    cy = (bmy0 + bmy1) / 2.0
    w = bmx1 - bmx0
    h = bmy1 - bmy0
    g0 = (cx - pcx) / (pw / 10.0)
    g1 = (cy - pcy) / (ph / 10.0)
    g2 = jnp.log(w / pw) * 5.0
    g3 = jnp.log(h / ph) * 5.0

    locs_b = locs_ref[0]
    loc_abs = (jnp.abs(locs_b[0:1] - g0) + jnp.abs(locs_b[1:2] - g1) +
               jnp.abs(locs_b[2:3] - g2) + jnp.abs(locs_b[3:4] - g3))
    loc_abs_row = jnp.sum(loc_abs * posf)

    l0 = scores_ref[0, 0:1]
    l1 = scores_ref[0, 1:2]
    l2 = scores_ref[0, 2:3]

    def _sp(l):
        return jnp.maximum(l, 0.0) + jnp.log1p(jnp.exp(-jnp.abs(l)))

    s_all = _sp(l0) + _sp(l1) + _sp(l2)
    d = jnp.where(tc == 0, l0,
                  jnp.where(tc == 1, l1,
                            jnp.where(tc == 2, l2, l1 + l2)))
    conf_all = s_all - d
    conf_pos_row = jnp.sum(conf_all * posf)
    conf_ref[0] = jnp.where(posm, 0.0, conf_all)

    lane = jax.lax.broadcasted_iota(jnp.int32, (1, 128), 1)
    stats_ref[0] = jnp.where(
        lane == 0, n_pos,
        jnp.where(lane == 1, conf_pos_row,
                  jnp.where(lane == 2, loc_abs_row, 0.0)))


def _stage2_body(conf_ref, stats_ref, out_ref):
    b, p = conf_ref.shape
    conf = conf_ref[...]
    stats = stats_ref[...]
    n_pos = stats[:, 0:1]
    npt = jnp.sum(n_pos)
    kf = jnp.minimum(n_pos * _NEG_POS_RATIO, float(p))

    bits = jax.lax.bitcast_convert_type(conf, jnp.int32)
    lo0 = jnp.full((b, 1), -1, jnp.int32)
    hi0 = jnp.full((b, 1), jnp.iinfo(jnp.int32).max - 1, jnp.int32)

    def _it(_, carry):
        lo, hi = carry
        mid = lo + ((hi - lo) >> 1)
        cnt = jnp.sum(jnp.where(bits > mid, 1.0, 0.0),
                      axis=1, keepdims=True)
        take_hi = cnt >= kf
        return jnp.where(take_hi, mid, lo), jnp.where(take_hi, hi, mid)

    _, hi = jax.lax.fori_loop(0, 31, _it, (lo0, hi0))
    vk_bits = hi
    vk = jax.lax.bitcast_convert_type(vk_bits, jnp.float32)
    gt = bits > vk_bits
    cnt_g = jnp.sum(jnp.where(gt, 1.0, 0.0), axis=1, keepdims=True)
    sum_g = jnp.sum(jnp.where(gt, conf, 0.0), axis=1, keepdims=True)
    hard = jnp.where(kf > 0.0, sum_g + vk * (kf - cnt_g), 0.0)

    conf_pos = jnp.sum(stats[:, 1:2])
    loc_abs = jnp.sum(stats[:, 2:3])
    conf_loss = (jnp.sum(hard) + conf_pos) / (1e-10 + npt)
    loc_loss = jnp.where(npt > 0.0,
                         loc_abs / (4.0 * jnp.maximum(npt, 1.0)), 0.0)
    total = conf_loss + _ALPHA * loc_loss

    lane = jax.lax.broadcasted_iota(jnp.int32, (1, 128), 1)
    out_ref[...] = jnp.where(
        lane == 0, total,
        jnp.where(lane == 1, conf_loss,
                  jnp.where(lane == 2, loc_loss, 0.0)))


def kernel(predicted_locs, predicted_scores, boxes, labels, priors_cxcy):
    bsz, p, _ = predicted_scores.shape
    n_obj = boxes.shape[1]

    locs_t = jnp.transpose(predicted_locs, (0, 2, 1))
    scores_t = jnp.transpose(predicted_scores, (0, 2, 1))
    priors_t = jnp.transpose(priors_cxcy, (1, 0))
    labels_c = labels.reshape(bsz, n_obj, 1)

    conf_neg, stats = pl.pallas_call(
        _stage1_body,
        grid=(bsz,),
        in_specs=[
            pl.BlockSpec((1, 4, p), lambda i: (i, 0, 0)),
            pl.BlockSpec((1, 3, p), lambda i: (i, 0, 0)),
            pl.BlockSpec((1, n_obj, 4), lambda i: (i, 0, 0)),
            pl.BlockSpec((1, n_obj, 1), lambda i: (i, 0, 0)),
            pl.BlockSpec((4, p), lambda i: (0, 0)),
        ],
        out_specs=[
            pl.BlockSpec((1, 1, p), lambda i: (i, 0, 0)),
            pl.BlockSpec((1, 1, 128), lambda i: (i, 0, 0)),
        ],
        out_shape=[
            jax.ShapeDtypeStruct((bsz, 1, p), jnp.float32),
            jax.ShapeDtypeStruct((bsz, 1, 128), jnp.float32),
        ],
    )(locs_t, scores_t, boxes, labels_c, priors_t)

    conf2 = conf_neg.reshape(bsz, p)
    stats2 = stats.reshape(bsz, 128)

    out = pl.pallas_call(
        _stage2_body,
        out_shape=jax.ShapeDtypeStruct((1, 128), jnp.float32),
    )(conf2, stats2)

    total = out[0, 0]
    conf_loss = out[0, 1]
    loc_loss = out[0, 2]
    n_positives = stats2[:, 0].astype(jnp.int32)
    return total, conf_loss, loc_loss, n_positives


# speedup vs baseline: 56.8107x; 1.2420x over previous
import jax
import jax.numpy as jnp
from jax.experimental import pallas as pl
from jax.experimental.pallas import tpu as pltpu

_THRESHOLD = 0.5
_NEG_POS_RATIO = 3.0
_ALPHA = 1.0
_SUB = 8


def _stage1_body(locs_ref, scores_ref, boxes_ref, labels_ref, priors_ref,
                 conf_ref, stats_ref, *, n_obj, p_real):
    s, l = priors_ref.shape[1], priors_ref.shape[2]
    p2 = s * l

    pcx = priors_ref[0]
    pcy = priors_ref[1]
    pw = priors_ref[2]
    ph = priors_ref[3]
    px0 = pcx - pw / 2.0
    py0 = pcy - ph / 2.0
    px1 = pcx + pw / 2.0
    py1 = pcy + ph / 2.0
    a2 = (px1 - px0) * (py1 - py0)

    p_idx = (jax.lax.broadcasted_iota(jnp.int32, (s, l), 0) * l +
             jax.lax.broadcasted_iota(jnp.int32, (s, l), 1))
    valid = p_idx < p_real

    ofp = jnp.full((s, l), -1.0, jnp.float32)
    obj_fp = jnp.zeros((s, l), jnp.int32)
    pfo = []
    for j in range(n_obj):
        bx0 = boxes_ref[0, j, 0]
        by0 = boxes_ref[0, j, 1]
        bx1 = boxes_ref[0, j, 2]
        by1 = boxes_ref[0, j, 3]
        iw = jnp.maximum(jnp.minimum(bx1, px1) - jnp.maximum(bx0, px0), 0.0)
        ih = jnp.maximum(jnp.minimum(by1, py1) - jnp.maximum(by0, py0), 0.0)
        inter = iw * ih
        a1 = (bx1 - bx0) * (by1 - by0)
        ov = inter / (a1 + a2 - inter)
        upd = ov > ofp
        obj_fp = jnp.where(upd, j, obj_fp)
        ofp = jnp.where(upd, ov, ofp)
        rmax = jnp.max(ov)
        pfo.append(jnp.min(jnp.where(ov >= rmax, p_idx, p2)))

    for j in range(n_obj):
        m = p_idx == pfo[j]
        obj_fp = jnp.where(m, j, obj_fp)
        ofp = jnp.where(m, 1.0, ofp)

    lab = jnp.zeros((s, l), jnp.int32)
    mx0 = jnp.zeros((s, l), jnp.float32)
    my0 = jnp.zeros((s, l), jnp.float32)
    mx1 = jnp.zeros((s, l), jnp.float32)
    my1 = jnp.zeros((s, l), jnp.float32)
    for j in range(n_obj):
        m = obj_fp == j
        lab = jnp.where(m, labels_ref[0, 0, j], lab)
        mx0 = jnp.where(m, boxes_ref[0, j, 0], mx0)
        my0 = jnp.where(m, boxes_ref[0, j, 1], my0)
        mx1 = jnp.where(m, boxes_ref[0, j, 2], mx1)
        my1 = jnp.where(m, boxes_ref[0, j, 3], my1)

    tc = jnp.where(ofp < _THRESHOLD, 0, lab)
    posm = (tc > 0) & valid
    posf = jnp.where(posm, 1.0, 0.0)
    n_pos = jnp.sum(posf)

    cx = (mx0 + mx1) / 2.0
    cy = (my0 + my1) / 2.0
    w = mx1 - mx0
    h = my1 - my0
    g0 = (cx - pcx) / (pw / 10.0)
    g1 = (cy - pcy) / (ph / 10.0)
    g2 = jnp.log(w / pw) * 5.0
    g3 = jnp.log(h / ph) * 5.0

    loc_abs = (jnp.abs(locs_ref[0, 0] - g0) + jnp.abs(locs_ref[0, 1] - g1) +
               jnp.abs(locs_ref[0, 2] - g2) + jnp.abs(locs_ref[0, 3] - g3))
    loc_abs_row = jnp.sum(jnp.where(posm, loc_abs, 0.0))

    l0 = scores_ref[0, 0]
    l1 = scores_ref[0, 1]
    l2 = scores_ref[0, 2]

    def _sp(x):
        return jnp.maximum(x, 0.0) + jnp.log1p(jnp.exp(-jnp.abs(x)))

    s_all = _sp(l0) + _sp(l1) + _sp(l2)
    d = jnp.where(tc == 0, l0,
                  jnp.where(tc == 1, l1,
                            jnp.where(tc == 2, l2, l1 + l2)))
    conf_all = s_all - d
    conf_pos_row = jnp.sum(jnp.where(posm, conf_all, 0.0))
    conf_ref[0] = jnp.where(posm | ~valid, 0.0, conf_all)

    lane = jax.lax.broadcasted_iota(jnp.int32, (1, 128), 1)
    stats_ref[0] = jnp.where(
        lane == 0, n_pos,
        jnp.where(lane == 1, conf_pos_row,
                  jnp.where(lane == 2, loc_abs_row, 0.0)))


def _stage2_body(conf_ref, stats_ref, out_ref, *, p_real):
    b = conf_ref.shape[0]
    conf = conf_ref[...]
    stats = stats_ref[...]
    n_pos = stats[:, 0:1]
    npt = jnp.sum(n_pos)
    kf = jnp.minimum(n_pos * _NEG_POS_RATIO, float(p_real))

    bits = jax.lax.bitcast_convert_type(conf, jnp.int32)
    lo0 = jnp.full((b, 1), -1, jnp.int32)
    hi0 = jnp.full((b, 1), jnp.iinfo(jnp.int32).max - 1, jnp.int32)

    def _it(_, carry):
        lo, hi = carry
        mid = lo + ((hi - lo) >> 1)
        cnt = jnp.sum(jnp.where(bits > mid, 1.0, 0.0),
                      axis=1, keepdims=True)
        take_hi = cnt >= kf
        return jnp.where(take_hi, mid, lo), jnp.where(take_hi, hi, mid)

    _, hi = jax.lax.fori_loop(0, 31, _it, (lo0, hi0))
    vk_bits = hi
    vk = jax.lax.bitcast_convert_type(vk_bits, jnp.float32)
    gt = bits > vk_bits
    cnt_g = jnp.sum(jnp.where(gt, 1.0, 0.0), axis=1, keepdims=True)
    sum_g = jnp.sum(jnp.where(gt, conf, 0.0), axis=1, keepdims=True)
    hard = jnp.where(kf > 0.0, sum_g + vk * (kf - cnt_g), 0.0)

    conf_pos = jnp.sum(stats[:, 1:2])
    loc_abs = jnp.sum(stats[:, 2:3])
    conf_loss = (jnp.sum(hard) + conf_pos) / (1e-10 + npt)
    loc_loss = jnp.where(npt > 0.0,
                         loc_abs / (4.0 * jnp.maximum(npt, 1.0)), 0.0)
    total = conf_loss + _ALPHA * loc_loss

    lane = jax.lax.broadcasted_iota(jnp.int32, (1, 128), 1)
    out_ref[...] = jnp.where(
        lane == 0, total,
        jnp.where(lane == 1, conf_loss,
                  jnp.where(lane == 2, loc_loss, 0.0)))


def kernel(predicted_locs, predicted_scores, boxes, labels, priors_cxcy):
    import functools

    bsz, p, _ = predicted_scores.shape
    n_obj = boxes.shape[1]
    lpad = -p % _SUB
    p2 = p + (-p % (_SUB * 128))
    lsz = p2 // _SUB

    def _fold(x_t):
        pads = [(0, 0)] * (x_t.ndim - 1) + [(0, p2 - p)]
        return jnp.pad(x_t, pads).reshape(x_t.shape[:-1] + (_SUB, lsz))

    del lpad
    locs_f = _fold(jnp.transpose(predicted_locs, (0, 2, 1)))
    scores_f = _fold(jnp.transpose(predicted_scores, (0, 2, 1)))
    priors_f = _fold(jnp.transpose(priors_cxcy, (1, 0)))

    conf_neg, stats = pl.pallas_call(
        functools.partial(_stage1_body, n_obj=n_obj, p_real=p),
        grid=(bsz,),
        in_specs=[
            pl.BlockSpec((1, 4, _SUB, lsz), lambda i: (i, 0, 0, 0)),
            pl.BlockSpec((1, 3, _SUB, lsz), lambda i: (i, 0, 0, 0)),
            pl.BlockSpec((1, n_obj, 4), lambda i: (i, 0, 0),
                         memory_space=pltpu.SMEM),
            pl.BlockSpec((1, 1, n_obj), lambda i: (i, 0, 0),
                         memory_space=pltpu.SMEM),
            pl.BlockSpec((4, _SUB, lsz), lambda i: (0, 0, 0)),
        ],
        out_specs=[
            pl.BlockSpec((1, _SUB, lsz), lambda i: (i, 0, 0)),
            pl.BlockSpec((1, 1, 128), lambda i: (i, 0, 0)),
        ],
        out_shape=[
            jax.ShapeDtypeStruct((bsz, _SUB, lsz), jnp.float32),
            jax.ShapeDtypeStruct((bsz, 1, 128), jnp.float32),
        ],
    )(locs_f, scores_f, boxes, labels.reshape(bsz, 1, n_obj), priors_f)

    conf2 = conf_neg.reshape(bsz, p2)
    stats2 = stats.reshape(bsz, 128)

    out = pl.pallas_call(
        functools.partial(_stage2_body, p_real=p),
        out_shape=jax.ShapeDtypeStruct((1, 128), jnp.float32),
    )(conf2, stats2)

    total = out[0, 0]
    conf_loss = out[0, 1]
    loc_loss = out[0, 2]
    n_positives = stats2[:, 0].astype(jnp.int32)
    return total, conf_loss, loc_loss, n_positives
